# baseline (speedup 1.0000x reference)
import jax
import jax.numpy as jnp
from jax.experimental import pallas as pl
from jax.experimental.pallas import tpu as pltpu

_K = 300
_KPAD = 384
_LANES = 128


def _select_kernel(logits_ref, idx_ref, s_ref):
    n = logits_ref.shape[1]
    rows = (n + _LANES - 1) // _LANES
    pad = rows * _LANES - n

    s = jnp.max(logits_ref[...], axis=2)
    s = jnp.concatenate(
        [s, jnp.full((1, pad), -jnp.inf, jnp.float32)], axis=1)
    s_ref[...] = s.reshape(rows, _LANES)

    riota = jax.lax.broadcasted_iota(jnp.int32, (1, rows), 1)
    liota = jax.lax.broadcasted_iota(jnp.int32, (1, _LANES), 1)
    rowmax0 = jnp.max(s_ref[...], axis=1).reshape(1, rows)

    def body(k, rowmax):
        m = jnp.max(rowmax)
        g = jnp.min(jnp.where(rowmax >= m, riota, rows))
        row = s_ref[pl.ds(g, 1), :]
        iloc = jnp.min(jnp.where(row >= m, liota, _LANES))
        idx_ref[0, 0, k] = g * _LANES + iloc
        row = jnp.where(liota == iloc, -jnp.inf, row)
        s_ref[pl.ds(g, 1), :] = row
        return jnp.where(riota == g, jnp.max(row), rowmax)

    jax.lax.fori_loop(0, _K, body, rowmax0)


def _gather_lc_kernel(idx_ref, logits_ref, coords_ref, tlog_ref, tcrd_ref):
    b = pl.program_id(0)

    def body(k, carry):
        i = idx_ref[b, 0, k]
        tlog_ref[0, pl.ds(k, 1), :] = logits_ref[0, pl.ds(i, 1), :]
        tcrd_ref[0, pl.ds(k, 1), :] = coords_ref[0, pl.ds(i, 1), :]
        return carry

    jax.lax.fori_loop(0, _K, body, 0)


def _gather_mem_kernel(idx_ref, mem_ref, out_ref):
    b = pl.program_id(0)

    def body(k, carry):
        i = idx_ref[b, 0, k]
        out_ref[0, pl.ds(k, 1), :] = mem_ref[0, pl.ds(i, 1), :]
        return carry

    jax.lax.fori_loop(0, _K, body, 0)


def kernel(memory, query_class_logits, query_geometries_unactivated):
    B, N, D = memory.shape
    C = query_class_logits.shape[-1]

    idx = pl.pallas_call(
        _select_kernel,
        grid=(B,),
        in_specs=[pl.BlockSpec((1, N, C), lambda b: (b, 0, 0))],
        out_specs=pl.BlockSpec((1, 1, _K), lambda b: (b, 0, 0),
                               memory_space=pltpu.SMEM),
        out_shape=jax.ShapeDtypeStruct((B, 1, _K), jnp.int32),
        scratch_shapes=[
            pltpu.VMEM(((N + _LANES - 1) // _LANES, _LANES), jnp.float32)],
        compiler_params=pltpu.CompilerParams(
            dimension_semantics=("parallel",)),
    )(query_class_logits)

    tlog, tcrd = pl.pallas_call(
        _gather_lc_kernel,
        grid_spec=pltpu.PrefetchScalarGridSpec(
            num_scalar_prefetch=1,
            grid=(B,),
            in_specs=[
                pl.BlockSpec((1, N, C), lambda b, idx: (b, 0, 0)),
                pl.BlockSpec((1, N, 4), lambda b, idx: (b, 0, 0)),
            ],
            out_specs=[
                pl.BlockSpec((1, _K, C), lambda b, idx: (b, 0, 0)),
                pl.BlockSpec((1, _K, 4), lambda b, idx: (b, 0, 0)),
            ],
        ),
        out_shape=[
            jax.ShapeDtypeStruct((B, _K, C), jnp.float32),
            jax.ShapeDtypeStruct((B, _K, 4), jnp.float32),
        ],
        compiler_params=pltpu.CompilerParams(
            dimension_semantics=("parallel",)),
    )(idx, query_class_logits, query_geometries_unactivated)

    tmem = pl.pallas_call(
        _gather_mem_kernel,
        grid_spec=pltpu.PrefetchScalarGridSpec(
            num_scalar_prefetch=1,
            grid=(B,),
            in_specs=[pl.BlockSpec((1, N, D), lambda b, idx: (b, 0, 0))],
            out_specs=pl.BlockSpec((1, _K, D), lambda b, idx: (b, 0, 0)),
        ),
        out_shape=jax.ShapeDtypeStruct((B, _K, D), jnp.float32),
        compiler_params=pltpu.CompilerParams(
            dimension_semantics=("parallel",)),
    )(idx, memory)

    return tmem, tlog, tcrd
